# baseline (speedup 1.0000x reference)
import functools

import jax
import jax.numpy as jnp
from jax import lax
from jax.experimental import pallas as pl
from jax.experimental.pallas import tpu as pltpu
from jax.experimental.pallas import tpu_sc as plsc

N = 10000
E = 320000
D = 128

NC = 2
NS = 16
NW = NC * NS

N_PAD = 10112
ROWS_PER_SUB = N_PAD // NS

CHUNK = 128
NCH = -(-E // (NW * CHUNK))
E_PAD = NW * NCH * CHUNK
NDUMMY = 3

HIST_PAD = 10240
DEG_SH = 2 * HIST_PAD
DEG_PER_SUB = HIST_PAD // NS

BR = 632
GRID = N_PAD // BR

_MESH = plsc.VectorSubcoreMesh(
    core_axis_name="c", subcore_axis_name="s", num_cores=NC, num_subcores=NS
)


def _wid():
    return lax.axis_index("s") * NC + lax.axis_index("c")


@functools.partial(
    pl.kernel,
    out_type=jax.ShapeDtypeStruct((NC, DEG_SH), jnp.float32),
    mesh=_MESH,
    scratch_types=[
        pltpu.VMEM((NCH + 1, CHUNK), jnp.int32),
        pltpu.VMEM((NCH + 1, CHUNK), jnp.int32),
        pltpu.VMEM((CHUNK,), jnp.float32),
        pltpu.VMEM_SHARED((HIST_PAD,), jnp.float32),
        pltpu.VMEM_SHARED((HIST_PAD,), jnp.float32),
        pltpu.SemaphoreType.DMA,
    ],
)
def _sc_degrees(row_hbm, col_hbm, zeros_hbm, out_hbm, row_v, col_v, ones_v,
                rhist, chist, sem):
    cid = lax.axis_index("c")
    sid = lax.axis_index("s")
    wid = _wid()

    pltpu.sync_copy(row_hbm.at[wid, pl.ds(0, NCH + 1)], row_v)
    pltpu.sync_copy(col_hbm.at[wid, pl.ds(0, NCH + 1)], col_v)
    for j in range(CHUNK // 16):
        ones_v[pl.ds(16 * j, 16)] = jnp.ones((16,), jnp.float32)

    sl = pl.ds(sid * DEG_PER_SUB, DEG_PER_SUB)
    pltpu.sync_copy(zeros_hbm.at[sl], rhist.at[sl])
    pltpu.sync_copy(zeros_hbm.at[sl], chist.at[sl])
    plsc.subcore_barrier()

    def pair(j):
        pltpu.async_copy(ones_v, rhist.at[row_v.at[j]], sem, add=True)
        pltpu.async_copy(ones_v, chist.at[col_v.at[j]], sem, add=True)

    def wait_pair():
        pltpu.make_async_copy(ones_v, rhist.at[pl.ds(0, CHUNK)], sem).wait()
        pltpu.make_async_copy(ones_v, chist.at[pl.ds(0, CHUNK)], sem).wait()

    for j in range(4):
        pair(j)

    def body(j, carry):
        wait_pair()
        pair(j)
        return carry

    lax.fori_loop(4, NCH, body, 0)
    for _ in range(4):
        wait_pair()
    plsc.subcore_barrier()

    pltpu.sync_copy(rhist.at[sl], out_hbm.at[cid, sl])
    pltpu.sync_copy(
        chist.at[sl], out_hbm.at[cid, pl.ds(HIST_PAD + sid * DEG_PER_SUB,
                                            DEG_PER_SUB)])


@functools.partial(
    pl.kernel,
    out_type=jax.ShapeDtypeStruct((NC * N_PAD, D), jnp.float32),
    mesh=_MESH,
    scratch_types=(
        [pltpu.VMEM((1, CHUNK), jnp.int32)] * 7
        + [
            pltpu.VMEM((CHUNK, D), jnp.float32),
            pltpu.VMEM((CHUNK, D), jnp.float32),
            pltpu.VMEM((CHUNK, D), jnp.float32),
            pltpu.VMEM_SHARED((N_PAD, D), jnp.float32),
            pltpu.SemaphoreType.DMA,
            pltpu.SemaphoreType.DMA,
            pltpu.SemaphoreType.DMA,
            pltpu.SemaphoreType.DMA,
        ]
    ),
)
def _sc_msg(xs_hbm, row_hbm, col_hbm, zeros_hbm, out_hbm,
            rib0, rib1, rib2, rib3, cib0, cib1, cib2,
            buf0, buf1, buf2, acc_sh, sem_r, sem_i, sem_g, sem_s):
    cid = lax.axis_index("c")
    sid = lax.axis_index("s")
    wid = _wid()
    bufs = (buf0, buf1, buf2)
    ribs = (rib0, rib1, rib2, rib3)
    cibs = (cib0, cib1, cib2)

    def fetch_row(k):
        pltpu.async_copy(row_hbm.at[wid, pl.ds(k, 1)], ribs[k % 4], sem_r)

    def wait_row():
        pltpu.make_async_copy(row_hbm.at[wid, pl.ds(0, 1)], rib0, sem_r).wait()

    def fetch_col(k):
        pltpu.async_copy(col_hbm.at[wid, pl.ds(k, 1)], cibs[k % 3], sem_i)

    def wait_col():
        pltpu.make_async_copy(col_hbm.at[wid, pl.ds(0, 1)], cib0, sem_i).wait()

    def gather(k):
        pltpu.async_copy(xs_hbm.at[cibs[k % 3].at[0]], bufs[k % 3], sem_g)

    def wait_gather():
        pltpu.make_async_copy(
            xs_hbm.at[pl.ds(0, CHUNK)], buf0, sem_g).wait()

    def scatter(k):
        pltpu.async_copy(
            bufs[k % 3], acc_sh.at[ribs[k % 4].at[0]], sem_s, add=True)

    def wait_scatter():
        pltpu.make_async_copy(
            buf0, acc_sh.at[pl.ds(0, CHUNK)], sem_s).wait()

    fetch_col(0)
    fetch_col(1)
    fetch_row(0)

    sl = pl.ds(sid * ROWS_PER_SUB, ROWS_PER_SUB)

    @pl.when(cid == 0)
    def _():
        pltpu.sync_copy(xs_hbm.at[sl], acc_sh.at[sl])

    @pl.when(cid != 0)
    def _():
        pltpu.sync_copy(zeros_hbm.at[sl], acc_sh.at[sl])

    plsc.subcore_barrier()

    wait_col()
    wait_row()
    gather(0)
    fetch_col(2)
    fetch_row(1)

    wait_col()
    wait_row()
    gather(1)
    wait_gather()
    fetch_col(3)
    scatter(0)
    fetch_row(2)

    wait_col()
    wait_row()
    gather(2)
    wait_gather()
    fetch_col(4)
    scatter(1)
    fetch_row(3)

    def tick(k):
        wait_scatter()
        fetch_row(k + 1)
        wait_col()
        wait_row()
        gather(k)
        wait_gather()
        fetch_col(k + 2)
        scatter(k - 1)

    def body(i, carry):
        for u in range(12):
            k = i * 12 + (3 + u)
            wait_scatter()
            pltpu.async_copy(
                row_hbm.at[wid, pl.ds(k + 1, 1)], ribs[u % 4], sem_r)
            wait_col()
            wait_row()
            pltpu.async_copy(
                xs_hbm.at[cibs[u % 3].at[0]], bufs[u % 3], sem_g)
            wait_gather()
            pltpu.async_copy(
                col_hbm.at[wid, pl.ds(k + 2, 1)], cibs[(2 + u) % 3], sem_i)
            pltpu.async_copy(
                bufs[(2 + u) % 3], acc_sh.at[ribs[(2 + u) % 4].at[0]],
                sem_s, add=True)
        return carry

    _un = (NCH - 3) // 12
    lax.fori_loop(0, _un, body, 0)
    for k in range(3 + 12 * _un, NCH):
        tick(k)

    wait_gather()
    scatter(NCH - 1)
    wait_scatter()
    wait_scatter()
    wait_scatter()
    wait_col()
    wait_col()
    wait_row()
    plsc.subcore_barrier()

    pltpu.sync_copy(
        acc_sh.at[sl],
        out_hbm.at[pl.ds(cid * N_PAD + sid * ROWS_PER_SUB, ROWS_PER_SUB)])


BR2 = 128
GRID2 = N_PAD // BR2


def _col(v, n):
    return lax.transpose(v.reshape(1, n), (1, 0))


def _tc_cols_body(dpo_ref, dpi_ref, d_ref, dinv_ref):
    deg_in = dpi_ref[0, :] + dpi_ref[1, :] + 1.0
    d_ref[...] = _col(lax.rsqrt(deg_in), BR2)
    deg_out = dpo_ref[0, :] + dpo_ref[1, :] + 1.0
    dinv_ref[...] = _col(1.0 / deg_out, BR2)


def _tc_cols(degp):
    return pl.pallas_call(
        _tc_cols_body,
        grid=(GRID2,),
        in_specs=[
            pl.BlockSpec((NC, BR2), lambda i: (0, i)),
            pl.BlockSpec((NC, BR2), lambda i: (0, i + HIST_PAD // BR2)),
        ],
        out_specs=[
            pl.BlockSpec((BR2, 1), lambda i: (i, 0)),
            pl.BlockSpec((BR2, 1), lambda i: (i, 0)),
        ],
        out_shape=[
            jax.ShapeDtypeStruct((N_PAD, 1), jnp.float32),
            jax.ShapeDtypeStruct((N_PAD, 1), jnp.float32),
        ],
    )(degp, degp)


def _tc_pre_body(h_ref, w_ref, d_ref, xs_ref):
    x = lax.dot_general(
        h_ref[...], w_ref[...], (((1,), (0,)), ((), ())),
        preferred_element_type=jnp.float32,
    )
    xs_ref[...] = x * d_ref[...]


def _tc_pre(h, W, d):
    return pl.pallas_call(
        _tc_pre_body,
        grid=(GRID,),
        in_specs=[
            pl.BlockSpec((BR, D), lambda i: (i, 0)),
            pl.BlockSpec((D, D), lambda i: (0, 0)),
            pl.BlockSpec((BR, 1), lambda i: (i, 0)),
        ],
        out_specs=pl.BlockSpec((BR, D), lambda i: (i, 0)),
        out_shape=jax.ShapeDtypeStruct((N_PAD, D), jnp.float32),
    )(h, W, d)


def _step_update(accp_ref, h_ref, d_ref, dinv_ref, b_ref, root_ref):
    acc = accp_ref[0] + accp_ref[1]
    h_msg = d_ref[...] * acc + b_ref[...]
    root_c = jax.nn.relu(h_ref[...] + root_ref[...]) * dinv_ref[...]
    return h_msg + root_c


def _tc_step_body(accp_ref, h_ref, d_ref, dinv_ref, b_ref, root_ref,
                  w_ref, hn_ref, xsn_ref):
    h_new = _step_update(accp_ref, h_ref, d_ref, dinv_ref, b_ref, root_ref)
    hn_ref[...] = h_new
    x = lax.dot_general(
        h_new, w_ref[...], (((1,), (0,)), ((), ())),
        preferred_element_type=jnp.float32,
    )
    xsn_ref[...] = x * d_ref[...]


def _tc_step(accp, h, d, dinv, b2, root, W):
    return pl.pallas_call(
        _tc_step_body,
        grid=(GRID,),
        in_specs=[
            pl.BlockSpec((NC, BR, D), lambda i: (0, i, 0)),
            pl.BlockSpec((BR, D), lambda i: (i, 0)),
            pl.BlockSpec((BR, 1), lambda i: (i, 0)),
            pl.BlockSpec((BR, 1), lambda i: (i, 0)),
            pl.BlockSpec((1, D), lambda i: (0, 0)),
            pl.BlockSpec((1, D), lambda i: (0, 0)),
            pl.BlockSpec((D, D), lambda i: (0, 0)),
        ],
        out_specs=[
            pl.BlockSpec((BR, D), lambda i: (i, 0)),
            pl.BlockSpec((BR, D), lambda i: (i, 0)),
        ],
        out_shape=[
            jax.ShapeDtypeStruct((N_PAD, D), jnp.float32),
            jax.ShapeDtypeStruct((N_PAD, D), jnp.float32),
        ],
    )(accp, h, d, dinv, b2, root, W)


def _tc_final_body(accp_ref, h_ref, d_ref, dinv_ref, b_ref, root_ref, hn_ref):
    hn_ref[...] = _step_update(accp_ref, h_ref, d_ref, dinv_ref, b_ref,
                               root_ref)


def _tc_final(accp, h, d, dinv, b2, root):
    return pl.pallas_call(
        _tc_final_body,
        grid=(GRID,),
        in_specs=[
            pl.BlockSpec((NC, BR, D), lambda i: (0, i, 0)),
            pl.BlockSpec((BR, D), lambda i: (i, 0)),
            pl.BlockSpec((BR, 1), lambda i: (i, 0)),
            pl.BlockSpec((BR, 1), lambda i: (i, 0)),
            pl.BlockSpec((1, D), lambda i: (0, 0)),
            pl.BlockSpec((1, D), lambda i: (0, 0)),
        ],
        out_specs=pl.BlockSpec((BR, D), lambda i: (i, 0)),
        out_shape=jax.ShapeDtypeStruct((N, D), jnp.float32),
    )(accp, h, d, dinv, b2, root)


def _pad_idx(idx, total):
    npad = total - idx.shape[0]
    sent = N + (jnp.arange(npad, dtype=jnp.int32) % (N_PAD - N))
    return jnp.concatenate([idx, sent])


@jax.jit
def kernel(in_feat, edge_index, W, b, root_emb):
    row = edge_index[0].astype(jnp.int32)
    col = edge_index[1].astype(jnp.int32)

    dummy = jnp.full((NW, NDUMMY, CHUNK), N, jnp.int32)
    row3 = jnp.concatenate(
        [_pad_idx(row, E_PAD).reshape(NW, NCH, CHUNK), dummy], axis=1)
    col3 = jnp.concatenate(
        [_pad_idx(col, E_PAD).reshape(NW, NCH, CHUNK), dummy], axis=1)

    zeros_feat = jnp.zeros((N_PAD, D), jnp.float32)
    zeros_deg = jnp.zeros((HIST_PAD,), jnp.float32)
    b2 = b.reshape(1, D)

    degp = _sc_degrees(row3, col3, zeros_deg)
    d, dinv = _tc_cols(degp)
    xs0 = _tc_pre(in_feat, W, d)

    accp0 = _sc_msg(xs0, row3, col3, zeros_feat).reshape(NC, N_PAD, D)
    h1, xs1 = _tc_step(accp0, in_feat, d, dinv, b2, root_emb, W)

    accp1 = _sc_msg(xs1, row3, col3, zeros_feat).reshape(NC, N_PAD, D)
    return _tc_final(accp1, h1, d, dinv, b2, root_emb)


# speedup vs baseline: 1.0315x; 1.0315x over previous
import functools

import jax
import jax.numpy as jnp
from jax import lax
from jax.experimental import pallas as pl
from jax.experimental.pallas import tpu as pltpu
from jax.experimental.pallas import tpu_sc as plsc

N = 10000
E = 320000
D = 128

NC = 2
NS = 16
NW = NC * NS

N_PAD = 10240
ROWS_PER_SUB = N_PAD // NS

CHUNK = 128
NCH = -(-E // (NW * CHUNK))
E_PAD = NW * NCH * CHUNK
NDUMMY = 3

DEG_SH = 2 * N_PAD
DEG_PER_SUB = N_PAD // NS

BR = 512
GRID = N_PAD // BR

_MESH = plsc.VectorSubcoreMesh(
    core_axis_name="c", subcore_axis_name="s", num_cores=NC, num_subcores=NS
)


def _wid():
    return lax.axis_index("s") * NC + lax.axis_index("c")


@functools.partial(
    pl.kernel,
    out_type=jax.ShapeDtypeStruct((NC, DEG_SH), jnp.float32),
    mesh=_MESH,
    scratch_types=[
        pltpu.VMEM((NCH + 1, CHUNK), jnp.int32),
        pltpu.VMEM((NCH + 1, CHUNK), jnp.int32),
        pltpu.VMEM((CHUNK,), jnp.float32),
        pltpu.VMEM_SHARED((N_PAD,), jnp.float32),
        pltpu.VMEM_SHARED((N_PAD,), jnp.float32),
        pltpu.SemaphoreType.DMA,
    ],
)
def _sc_degrees(row_hbm, col_hbm, zeros_hbm, out_hbm, row_v, col_v, ones_v,
                rhist, chist, sem):
    cid = lax.axis_index("c")
    sid = lax.axis_index("s")
    wid = _wid()

    pltpu.sync_copy(row_hbm.at[wid, pl.ds(0, NCH + 1)], row_v)
    pltpu.sync_copy(col_hbm.at[wid, pl.ds(0, NCH + 1)], col_v)
    for j in range(CHUNK // 16):
        ones_v[pl.ds(16 * j, 16)] = jnp.ones((16,), jnp.float32)

    sl = pl.ds(sid * DEG_PER_SUB, DEG_PER_SUB)
    pltpu.sync_copy(zeros_hbm.at[sl], rhist.at[sl])
    pltpu.sync_copy(zeros_hbm.at[sl], chist.at[sl])
    plsc.subcore_barrier()

    def pair(j):
        pltpu.async_copy(ones_v, rhist.at[row_v.at[j]], sem, add=True)
        pltpu.async_copy(ones_v, chist.at[col_v.at[j]], sem, add=True)

    def wait_pair():
        pltpu.make_async_copy(ones_v, rhist.at[pl.ds(0, CHUNK)], sem).wait()
        pltpu.make_async_copy(ones_v, chist.at[pl.ds(0, CHUNK)], sem).wait()

    for j in range(4):
        pair(j)

    def body(j, carry):
        wait_pair()
        pair(j)
        return carry

    lax.fori_loop(4, NCH, body, 0)
    for _ in range(4):
        wait_pair()
    plsc.subcore_barrier()

    pltpu.sync_copy(rhist.at[sl], out_hbm.at[cid, sl])
    pltpu.sync_copy(
        chist.at[sl], out_hbm.at[cid, pl.ds(N_PAD + sid * DEG_PER_SUB,
                                            DEG_PER_SUB)])


@functools.partial(
    pl.kernel,
    out_type=jax.ShapeDtypeStruct((NC * N_PAD, D), jnp.float32),
    mesh=_MESH,
    scratch_types=(
        [pltpu.VMEM((1, CHUNK), jnp.int32)] * 6
        + [
            pltpu.VMEM((NCH + 1, CHUNK), jnp.int32),
            pltpu.VMEM((CHUNK, D), jnp.float32),
            pltpu.VMEM((CHUNK, D), jnp.float32),
            pltpu.VMEM_SHARED((N_PAD, D), jnp.float32),
            pltpu.SemaphoreType.DMA,
            pltpu.SemaphoreType.DMA,
            pltpu.SemaphoreType.DMA,
        ]
    ),
)
def _sc_msg(xs_hbm, row_hbm, col_hbm, zeros_hbm, out_hbm,
            rib0, rib1, rib2, rib3, rib4, rib5,
            col_v, buf0, buf1, acc_sh, sem_r, sem_g, sem_s):
    cid = lax.axis_index("c")
    sid = lax.axis_index("s")
    wid = _wid()
    bufs = (buf0, buf1)
    ribs = (rib0, rib1, rib2, rib3, rib4, rib5)

    def fetch_idx(k):
        pltpu.async_copy(row_hbm.at[wid, pl.ds(k, 1)], ribs[k % 6], sem_r)

    def wait_idx():
        pltpu.make_async_copy(row_hbm.at[wid, pl.ds(0, 1)], rib0, sem_r).wait()

    def gather(k):
        pltpu.async_copy(xs_hbm.at[col_v.at[k]], bufs[k % 2], sem_g)

    def wait_gather():
        pltpu.make_async_copy(
            xs_hbm.at[pl.ds(0, CHUNK)], buf0, sem_g).wait()

    def scatter(k):
        pltpu.async_copy(
            bufs[k % 2], acc_sh.at[ribs[k % 6].at[0]], sem_s, add=True)

    def wait_scatter():
        pltpu.make_async_copy(
            buf0, acc_sh.at[pl.ds(0, CHUNK)], sem_s).wait()

    fetch_idx(0)
    fetch_idx(1)
    fetch_idx(2)
    pltpu.sync_copy(col_hbm.at[wid, pl.ds(0, NCH + 1)], col_v)

    sl = pl.ds(sid * ROWS_PER_SUB, ROWS_PER_SUB)

    @pl.when(cid == 0)
    def _():
        pltpu.sync_copy(xs_hbm.at[sl], acc_sh.at[sl])

    @pl.when(cid != 0)
    def _():
        pltpu.sync_copy(zeros_hbm.at[sl], acc_sh.at[sl])

    plsc.subcore_barrier()

    wait_idx()
    gather(0)
    fetch_idx(3)
    wait_idx()
    gather(1)
    wait_gather()
    scatter(0)
    fetch_idx(4)

    def tick(k):
        wait_scatter()
        fetch_idx(k + 3)
        wait_idx()
        gather(k)
        wait_gather()
        scatter(k - 1)

    def body(i, carry):
        for u in range(6):
            k = i * 6 + (2 + u)
            wait_scatter()
            pltpu.async_copy(
                row_hbm.at[wid, pl.ds(k + 3, 1)], ribs[(5 + u) % 6], sem_r)
            wait_idx()
            pltpu.async_copy(
                xs_hbm.at[col_v.at[k]], bufs[u % 2], sem_g)
            wait_gather()
            pltpu.async_copy(
                bufs[(1 + u) % 2], acc_sh.at[ribs[(1 + u) % 6].at[0]],
                sem_s, add=True)
        return carry

    _un = (NCH - 2) // 6
    lax.fori_loop(0, _un, body, 0)
    for k in range(2 + 6 * _un, NCH):
        tick(k)

    wait_gather()
    scatter(NCH - 1)
    wait_scatter()
    wait_scatter()
    wait_idx()
    wait_idx()
    wait_idx()
    plsc.subcore_barrier()

    pltpu.sync_copy(
        acc_sh.at[sl],
        out_hbm.at[pl.ds(cid * N_PAD + sid * ROWS_PER_SUB, ROWS_PER_SUB)])


def _col(v):
    return lax.transpose(v.reshape(1, BR), (1, 0))


def _deg_cols(dpo_ref, dpi_ref):
    deg_in = dpi_ref[0, :] + dpi_ref[1, :] + 1.0
    dcol = _col(lax.rsqrt(deg_in))
    deg_out = dpo_ref[0, :] + dpo_ref[1, :] + 1.0
    dinvcol = _col(1.0 / deg_out)
    return dcol, dinvcol


def _tc_pre_body(h_ref, w_ref, dpo_ref, dpi_ref, xs_ref, d_ref, dinv_ref):
    dcol, dinvcol = _deg_cols(dpo_ref, dpi_ref)
    x = lax.dot_general(
        h_ref[...], w_ref[...], (((1,), (0,)), ((), ())),
        preferred_element_type=jnp.float32,
    )
    xs_ref[...] = x * dcol
    d_ref[...] = dcol
    dinv_ref[...] = dinvcol


def _tc_pre(h, W, degp):
    return pl.pallas_call(
        _tc_pre_body,
        grid=(GRID,),
        in_specs=[
            pl.BlockSpec((BR, D), lambda i: (i, 0)),
            pl.BlockSpec((D, D), lambda i: (0, 0)),
            pl.BlockSpec((NC, BR), lambda i: (0, i)),
            pl.BlockSpec((NC, BR), lambda i: (0, i + GRID)),
        ],
        out_specs=[
            pl.BlockSpec((BR, D), lambda i: (i, 0)),
            pl.BlockSpec((BR, 1), lambda i: (i, 0)),
            pl.BlockSpec((BR, 1), lambda i: (i, 0)),
        ],
        out_shape=[
            jax.ShapeDtypeStruct((N_PAD, D), jnp.float32),
            jax.ShapeDtypeStruct((N_PAD, 1), jnp.float32),
            jax.ShapeDtypeStruct((N_PAD, 1), jnp.float32),
        ],
    )(h, W, degp, degp)


def _step_update(accp_ref, h_ref, d_ref, dinv_ref, b_ref, root_ref):
    acc = accp_ref[0] + accp_ref[1]
    h_msg = d_ref[...] * acc + b_ref[...]
    root_c = jax.nn.relu(h_ref[...] + root_ref[...]) * dinv_ref[...]
    return h_msg + root_c


def _tc_step_body(accp_ref, h_ref, d_ref, dinv_ref, b_ref, root_ref,
                  w_ref, hn_ref, xsn_ref):
    h_new = _step_update(accp_ref, h_ref, d_ref, dinv_ref, b_ref, root_ref)
    hn_ref[...] = h_new
    x = lax.dot_general(
        h_new, w_ref[...], (((1,), (0,)), ((), ())),
        preferred_element_type=jnp.float32,
    )
    xsn_ref[...] = x * d_ref[...]


def _tc_step(accp, h, d, dinv, b2, root, W):
    return pl.pallas_call(
        _tc_step_body,
        grid=(GRID,),
        in_specs=[
            pl.BlockSpec((NC, BR, D), lambda i: (0, i, 0)),
            pl.BlockSpec((BR, D), lambda i: (i, 0)),
            pl.BlockSpec((BR, 1), lambda i: (i, 0)),
            pl.BlockSpec((BR, 1), lambda i: (i, 0)),
            pl.BlockSpec((1, D), lambda i: (0, 0)),
            pl.BlockSpec((1, D), lambda i: (0, 0)),
            pl.BlockSpec((D, D), lambda i: (0, 0)),
        ],
        out_specs=[
            pl.BlockSpec((BR, D), lambda i: (i, 0)),
            pl.BlockSpec((BR, D), lambda i: (i, 0)),
        ],
        out_shape=[
            jax.ShapeDtypeStruct((N_PAD, D), jnp.float32),
            jax.ShapeDtypeStruct((N_PAD, D), jnp.float32),
        ],
    )(accp, h, d, dinv, b2, root, W)


def _tc_final_body(accp_ref, h_ref, d_ref, dinv_ref, b_ref, root_ref, hn_ref):
    hn_ref[...] = _step_update(accp_ref, h_ref, d_ref, dinv_ref, b_ref,
                               root_ref)


def _tc_final(accp, h, d, dinv, b2, root):
    return pl.pallas_call(
        _tc_final_body,
        grid=(GRID,),
        in_specs=[
            pl.BlockSpec((NC, BR, D), lambda i: (0, i, 0)),
            pl.BlockSpec((BR, D), lambda i: (i, 0)),
            pl.BlockSpec((BR, 1), lambda i: (i, 0)),
            pl.BlockSpec((BR, 1), lambda i: (i, 0)),
            pl.BlockSpec((1, D), lambda i: (0, 0)),
            pl.BlockSpec((1, D), lambda i: (0, 0)),
        ],
        out_specs=pl.BlockSpec((BR, D), lambda i: (i, 0)),
        out_shape=jax.ShapeDtypeStruct((N, D), jnp.float32),
    )(accp, h, d, dinv, b2, root)


def _pad_idx(idx, total):
    npad = total - idx.shape[0]
    sent = N + (jnp.arange(npad, dtype=jnp.int32) % (N_PAD - N))
    return jnp.concatenate([idx, sent])


@jax.jit
def kernel(in_feat, edge_index, W, b, root_emb):
    row = edge_index[0].astype(jnp.int32)
    col = edge_index[1].astype(jnp.int32)

    dummy = jnp.full((NW, NDUMMY, CHUNK), N, jnp.int32)
    row3 = jnp.concatenate(
        [_pad_idx(row, E_PAD).reshape(NW, NCH, CHUNK), dummy], axis=1)
    col3 = jnp.concatenate(
        [_pad_idx(col, E_PAD).reshape(NW, NCH, CHUNK), dummy], axis=1)

    zeros_feat = jnp.zeros((N_PAD, D), jnp.float32)
    zeros_deg = jnp.zeros((N_PAD,), jnp.float32)
    b2 = b.reshape(1, D)

    degp = _sc_degrees(row3, col3, zeros_deg)
    xs0, d, dinv = _tc_pre(in_feat, W, degp)

    accp0 = _sc_msg(xs0, row3, col3, zeros_feat).reshape(NC, N_PAD, D)
    h1, xs1 = _tc_step(accp0, in_feat, d, dinv, b2, root_emb, W)

    accp1 = _sc_msg(xs1, row3, col3, zeros_feat).reshape(NC, N_PAD, D)
    return _tc_final(accp1, h1, d, dinv, b2, root_emb)
